# baseline (speedup 1.0000x reference)
import functools

import jax
import jax.numpy as jnp
from jax import lax
from jax.experimental import pallas as pl
from jax.experimental.pallas import tpu as pltpu
from jax.experimental.pallas import tpu_sc as plsc

N_NODES = 10000
E_EDGES = 320000
D = 128

NC, NS = 2, 16
NW = NC * NS
CH = 128
EPT = 10112
E_PAD = EPT * NW
NCHUNK = EPT // CH
CHM = 128
C0 = 63
C1 = 95
N_PAD = 10240
ROWS_PT = N_PAD // NS
RB = 1024
GRID = N_PAD // RB

_sc_mesh = plsc.VectorSubcoreMesh(core_axis_name="c", subcore_axis_name="s")


@functools.partial(
    pl.kernel,
    out_type=(
        jax.ShapeDtypeStruct((NC * 3 * N_PAD,), jnp.float32),
        jax.ShapeDtypeStruct((E_PAD,), jnp.int32),
    ),
    mesh=_sc_mesh,
    compiler_params=pltpu.CompilerParams(needs_layout_passes=False),
    scratch_types=(
        pltpu.VMEM((CH,), jnp.int32),
        pltpu.VMEM((CH,), jnp.int32),
        pltpu.VMEM((CH,), jnp.int32),
        pltpu.VMEM((N_PAD,), jnp.float32),
        pltpu.VMEM((N_PAD,), jnp.float32),
        pltpu.VMEM((N_PAD,), jnp.float32),
        pltpu.VMEM((NS * ROWS_PT,), jnp.float32),
        pltpu.VMEM((ROWS_PT,), jnp.float32),
        pltpu.VMEM_SHARED((NS * N_PAD,), jnp.float32),
        pltpu.VMEM_SHARED((NS * N_PAD,), jnp.float32),
        pltpu.VMEM_SHARED((NS * N_PAD,), jnp.float32),
    ),
)
def _sc_degree(srcp, dstp, zeros1, deg_out, mdst_out,
               sbuf, dbuf, mdbuf, hist0, hist1, hist2, cbuf, res,
               sh0, sh1, sh2):
    c = lax.axis_index("c")
    t = lax.axis_index("s")
    w = c * NS + t
    pltpu.sync_copy(zeros1, hist0)
    pltpu.sync_copy(zeros1, hist1)
    pltpu.sync_copy(zeros1, hist2)
    onev = jnp.ones((16,), jnp.float32)

    def body(j, carry):
        base = w * EPT + j * CH
        pltpu.sync_copy(srcp.at[pl.ds(base, CH)], sbuf)
        pltpu.sync_copy(dstp.at[pl.ds(base, CH)], dbuf)
        for v in range(CH // 16):
            sl = pl.ds(v * 16, 16)
            sv = sbuf[sl]
            dv = dbuf[sl]
            m = sv != dv
            msv = jnp.where(m, sv, N_NODES)
            mdv = jnp.where(m, dv, N_NODES)
            mdbuf[sl] = mdv
            plsc.addupdate_scatter(hist0, [dv], onev)
            plsc.addupdate_scatter(hist1, [msv], onev)
            plsc.addupdate_scatter(hist2, [mdv], onev)
        pltpu.sync_copy(mdbuf, mdst_out.at[pl.ds(base, CH)])
        return carry

    lax.fori_loop(0, NCHUNK, body, 0)

    pltpu.sync_copy(hist0, sh0.at[pl.ds(t * N_PAD, N_PAD)])
    pltpu.sync_copy(hist1, sh1.at[pl.ds(t * N_PAD, N_PAD)])
    pltpu.sync_copy(hist2, sh2.at[pl.ds(t * N_PAD, N_PAD)])
    plsc.subcore_barrier()
    for k, sh in enumerate((sh0, sh1, sh2)):
        for r in range(NS):
            pltpu.sync_copy(sh.at[pl.ds(r * N_PAD + t * ROWS_PT, ROWS_PT)],
                            cbuf.at[pl.ds(r * ROWS_PT, ROWS_PT)])

        def comb(i, carry):
            s = cbuf[pl.ds(i * 16, 16)]
            for r in range(1, NS):
                s = s + cbuf[pl.ds(r * ROWS_PT + i * 16, 16)]
            res[pl.ds(i * 16, 16)] = s
            return carry

        lax.fori_loop(0, ROWS_PT // 16, comb, 0)
        pltpu.sync_copy(
            res,
            deg_out.at[pl.ds((c * 3 + k) * N_PAD + t * ROWS_PT, ROWS_PT)])


@functools.partial(
    pl.kernel,
    out_type=jax.ShapeDtypeStruct((NC, N_PAD, D), jnp.float32),
    mesh=_sc_mesh,
    scratch_types=(
        pltpu.VMEM((CHM,), jnp.int32),
        pltpu.VMEM((CHM,), jnp.int32),
        pltpu.VMEM((CHM, D), jnp.float32),
        pltpu.VMEM_SHARED((N_PAD, D), jnp.float32),
        pltpu.SemaphoreType.DMA,
    ),
)
def _sc_scatter(feat, isrc, idst, zerosd, out, sbuf, dbuf, rows_v, acc, sem):
    c = lax.axis_index("c")
    t = lax.axis_index("s")
    rows = pl.ds(t * ROWS_PT, ROWS_PT)
    nch = jnp.where(c == 0, C0, C1)
    ebase = jnp.where(c == 0, t * (C0 * CHM),
                      NS * C0 * CHM + t * (C1 * CHM))
    pltpu.sync_copy(zerosd.at[rows], acc.at[rows])
    plsc.subcore_barrier()

    def body(j, carry):
        base = ebase + j * CHM
        pltpu.sync_copy(isrc.at[pl.ds(base, CHM)], sbuf)
        pltpu.sync_copy(idst.at[pl.ds(base, CHM)], dbuf)
        pltpu.async_copy(feat.at[sbuf], rows_v, sem).wait()
        pltpu.sync_copy(rows_v, acc.at[dbuf], add=True)
        return carry

    lax.fori_loop(0, nch, body, 0)
    plsc.subcore_barrier()
    pltpu.sync_copy(acc.at[rows], out.at[c, rows])


def _rspec(imap=None):
    return pl.BlockSpec((RB, D), imap or (lambda i: (i, 0)))


def _wspec(r, c=D):
    return pl.BlockSpec((r, c), lambda i: (0, 0))


def _bspec():
    return pl.BlockSpec((D,), lambda i: (0,))


def _aspec():
    return pl.BlockSpec((NC, RB, D), lambda i: (0, i, 0))


_F32 = functools.partial(jnp.dot, preferred_element_type=jnp.float32)


def _stem_body(x_ref, dinv_ref, nout_ref, w1_ref, b1_ref, w2_ref, b2_ref,
               wg1_ref, h_ref, hh1_ref, fg1_ref):
    x = x_ref[...]
    h1 = jnp.maximum(_F32(x, w1_ref[...]) + b1_ref[...], 0.0)
    h = jnp.maximum(_F32(h1, w2_ref[...]) + b2_ref[...], 0.0)
    h_ref[...] = h
    hh1_ref[...] = h * dinv_ref[...]
    fg1_ref[...] = _F32(x * nout_ref[...], wg1_ref[...])


def _mid1_body(h_ref, a_ref, dinv_ref, l1_ref, hh2_ref):
    dinv = dinv_ref[...]
    l1 = h_ref[...] - (a_ref[0] + a_ref[1]) * dinv
    l1_ref[...] = l1
    hh2_ref[...] = l1 * dinv


def _band_body(h_ref, l1_ref, a2_ref, dinv_ref, w3_ref, b3_ref, w4_ref,
               b4_ref, out_ref):
    l1 = l1_ref[...]
    l2 = l1 - (a2_ref[0] + a2_ref[1]) * dinv_ref[...]
    w3 = w3_ref[...]
    wa = 3.0 * w3[0:128]
    wb = -3.0 * w3[0:128] + 3.0 * w3[128:256] + w3[384:512]
    wc = (0.75 * w3[0:128] - 1.5 * w3[128:256] + 0.75 * w3[256:384]
          + w3[512:640])
    acc = _F32(h_ref[...], wa) + _F32(l1, wb) + _F32(l2, wc) + b3_ref[...]
    out_ref[...] = _F32(jnp.maximum(acc, 0.0), w4_ref[...]) + b4_ref[...]


def _gmid_body(g1_ref, nin_ref, nout_ref, bg1_ref, wg2_ref, fg2_ref):
    emb1 = jnp.maximum((g1_ref[0] + g1_ref[1]) * nin_ref[...] + bg1_ref[...],
                       0.0)
    fg2_ref[...] = _F32(emb1 * nout_ref[...], wg2_ref[...])


def _gout_body(g2_ref, nin_ref, bg2_ref, emb_ref):
    emb_ref[...] = (g2_ref[0] + g2_ref[1]) * nin_ref[...] + bg2_ref[...]


def _nd(n_out=1):
    shp = jax.ShapeDtypeStruct((N_PAD, D), jnp.float32)
    return shp if n_out == 1 else (shp,) * n_out


def kernel(in_feat, edge_index, W1, b1, W2, b2, W3, b3, W4, b4,
           Wg1, bg1, Wg2, bg2):
    pad_idx = jnp.full((E_PAD - E_EDGES,), N_NODES, jnp.int32)
    srcp = jnp.concatenate([edge_index[0], pad_idx])
    dstp = jnp.concatenate([edge_index[1], pad_idx])
    x_pad = jnp.pad(in_feat, ((0, N_PAD - N_NODES), (0, 0)))
    zeros1 = jnp.zeros((N_PAD,), jnp.float32)
    zerosd = jnp.zeros((N_PAD, D), jnp.float32)
    w4p = jnp.pad(W4, ((0, 0), (0, D - W4.shape[1])))
    b4p = jnp.pad(b4, (0, D - b4.shape[0]))

    deg_raw, mdst = _sc_degree(srcp, dstp, zeros1)
    deg = deg_raw.reshape(NC, 3, N_PAD).sum(axis=0)

    scale = lax.rsqrt(jnp.maximum(deg, 1.0))
    dinvb = jnp.broadcast_to(scale[0][:, None], (N_PAD, D))
    noutb = jnp.broadcast_to(scale[1][:, None], (N_PAD, D))
    ninb = jnp.broadcast_to(scale[2][:, None], (N_PAD, D))

    h, hh1, fg1 = pl.pallas_call(
        _stem_body,
        grid=(GRID,),
        in_specs=[_rspec(), _rspec(), _rspec(), _wspec(D), _bspec(),
                  _wspec(D), _bspec(), _wspec(D)],
        out_specs=[_rspec(), _rspec(), _rspec()],
        out_shape=[_nd(), _nd(), _nd()],
    )(x_pad, dinvb, noutb, W1, b1, W2, b2, Wg1)

    a1 = _sc_scatter(hh1, srcp, dstp, zerosd)
    l1, hh2 = pl.pallas_call(
        _mid1_body,
        grid=(GRID,),
        in_specs=[_rspec(), _aspec(), _rspec()],
        out_specs=[_rspec(), _rspec()],
        out_shape=[_nd(), _nd()],
    )(h, a1, dinvb)

    a2 = _sc_scatter(hh2, srcp, dstp, zerosd)
    band = pl.pallas_call(
        _band_body,
        grid=(GRID,),
        in_specs=[_rspec(), _rspec(), _aspec(), _rspec(),
                  pl.BlockSpec((5 * D, D), lambda i: (0, 0)), _bspec(),
                  _wspec(D), _bspec()],
        out_specs=_rspec(),
        out_shape=_nd(),
    )(h, l1, a2, dinvb, W3, b3, w4p, b4p)

    g1 = _sc_scatter(fg1, srcp, mdst, zerosd)
    fg2 = pl.pallas_call(
        _gmid_body,
        grid=(GRID,),
        in_specs=[_aspec(), _rspec(), _rspec(), _bspec(), _wspec(D)],
        out_specs=_rspec(),
        out_shape=_nd(),
    )(g1, ninb, noutb, bg1, Wg2)

    g2 = _sc_scatter(fg2, srcp, mdst, zerosd)
    emb = pl.pallas_call(
        _gout_body,
        grid=(GRID,),
        in_specs=[_aspec(), _rspec(), _bspec()],
        out_specs=_rspec(),
        out_shape=_nd(),
    )(g2, ninb, bg2)

    return band[:N_NODES, :2], emb[:N_NODES]


# speedup vs baseline: 1.1946x; 1.1946x over previous
import functools

import jax
import jax.numpy as jnp
from jax import lax
from jax.experimental import pallas as pl
from jax.experimental.pallas import tpu as pltpu
from jax.experimental.pallas import tpu_sc as plsc

N_NODES = 10000
E_EDGES = 320000
D = 128

NC, NS = 2, 16
NW = NC * NS
CH = 128
EPT = 10112
E_PAD = EPT * NW
NCHUNK = EPT // CH
CHM = 128
C0 = 95
C1 = 63
N_PAD = 10240
ROWS_PT = N_PAD // NS
RB = 1024
GRID = N_PAD // RB

_sc_mesh = plsc.VectorSubcoreMesh(core_axis_name="c", subcore_axis_name="s")


@functools.partial(
    pl.kernel,
    out_type=(
        jax.ShapeDtypeStruct((NC * 3 * N_PAD,), jnp.float32),
        jax.ShapeDtypeStruct((E_PAD,), jnp.int32),
    ),
    mesh=_sc_mesh,
    compiler_params=pltpu.CompilerParams(needs_layout_passes=False),
    scratch_types=(
        pltpu.VMEM((CH,), jnp.int32),
        pltpu.VMEM((CH,), jnp.int32),
        pltpu.VMEM((CH,), jnp.int32),
        pltpu.VMEM((N_PAD,), jnp.float32),
        pltpu.VMEM((N_PAD,), jnp.float32),
        pltpu.VMEM((N_PAD,), jnp.float32),
        pltpu.VMEM((NS * ROWS_PT,), jnp.float32),
        pltpu.VMEM((ROWS_PT,), jnp.float32),
        pltpu.VMEM_SHARED((NS * N_PAD,), jnp.float32),
        pltpu.VMEM_SHARED((NS * N_PAD,), jnp.float32),
        pltpu.VMEM_SHARED((NS * N_PAD,), jnp.float32),
    ),
)
def _sc_degree(srcp, dstp, zeros1, deg_out, mdst_out,
               sbuf, dbuf, mdbuf, hist0, hist1, hist2, cbuf, res,
               sh0, sh1, sh2):
    c = lax.axis_index("c")
    t = lax.axis_index("s")
    w = c * NS + t
    pltpu.sync_copy(zeros1, hist0)
    pltpu.sync_copy(zeros1, hist1)
    pltpu.sync_copy(zeros1, hist2)
    onev = jnp.ones((16,), jnp.float32)

    def body(j, carry):
        base = w * EPT + j * CH
        pltpu.sync_copy(srcp.at[pl.ds(base, CH)], sbuf)
        pltpu.sync_copy(dstp.at[pl.ds(base, CH)], dbuf)
        for v in range(CH // 16):
            sl = pl.ds(v * 16, 16)
            sv = sbuf[sl]
            dv = dbuf[sl]
            m = sv != dv
            msv = jnp.where(m, sv, N_NODES)
            mdv = jnp.where(m, dv, N_NODES)
            mdbuf[sl] = mdv
            plsc.addupdate_scatter(hist0, [dv], onev)
            plsc.addupdate_scatter(hist1, [msv], onev)
            plsc.addupdate_scatter(hist2, [mdv], onev)
        pltpu.sync_copy(mdbuf, mdst_out.at[pl.ds(base, CH)])
        return carry

    lax.fori_loop(0, NCHUNK, body, 0)

    pltpu.sync_copy(hist0, sh0.at[pl.ds(t * N_PAD, N_PAD)])
    pltpu.sync_copy(hist1, sh1.at[pl.ds(t * N_PAD, N_PAD)])
    pltpu.sync_copy(hist2, sh2.at[pl.ds(t * N_PAD, N_PAD)])
    plsc.subcore_barrier()
    for k, sh in enumerate((sh0, sh1, sh2)):
        for r in range(NS):
            pltpu.sync_copy(sh.at[pl.ds(r * N_PAD + t * ROWS_PT, ROWS_PT)],
                            cbuf.at[pl.ds(r * ROWS_PT, ROWS_PT)])

        def comb(i, carry):
            s = cbuf[pl.ds(i * 16, 16)]
            for r in range(1, NS):
                s = s + cbuf[pl.ds(r * ROWS_PT + i * 16, 16)]
            res[pl.ds(i * 16, 16)] = s
            return carry

        lax.fori_loop(0, ROWS_PT // 16, comb, 0)
        pltpu.sync_copy(
            res,
            deg_out.at[pl.ds((c * 3 + k) * N_PAD + t * ROWS_PT, ROWS_PT)])


@functools.partial(
    pl.kernel,
    out_type=jax.ShapeDtypeStruct((NC, N_PAD, D), jnp.float32),
    mesh=_sc_mesh,
    scratch_types=(
        pltpu.VMEM((CHM,), jnp.int32),
        pltpu.VMEM((CHM,), jnp.int32),
        pltpu.VMEM((CHM, D), jnp.float32),
        pltpu.VMEM_SHARED((N_PAD, D), jnp.float32),
        pltpu.SemaphoreType.DMA,
    ),
)
def _sc_scatter(feat, isrc, idst, zerosd, out, sbuf, dbuf, rows_v, acc, sem):
    c = lax.axis_index("c")
    t = lax.axis_index("s")
    rows = pl.ds(t * ROWS_PT, ROWS_PT)
    nch = jnp.where(c == 0, C0, C1)
    ebase = jnp.where(c == 0, t * (C0 * CHM),
                      NS * C0 * CHM + t * (C1 * CHM))
    pltpu.sync_copy(zerosd.at[rows], acc.at[rows])
    plsc.subcore_barrier()

    def body(j, carry):
        base = ebase + j * CHM
        pltpu.sync_copy(isrc.at[pl.ds(base, CHM)], sbuf)
        pltpu.sync_copy(idst.at[pl.ds(base, CHM)], dbuf)
        pltpu.async_copy(feat.at[sbuf], rows_v, sem).wait()
        pltpu.sync_copy(rows_v, acc.at[dbuf], add=True)
        return carry

    lax.fori_loop(0, nch, body, 0)
    plsc.subcore_barrier()
    pltpu.sync_copy(acc.at[rows], out.at[c, rows])


def _rspec(imap=None):
    return pl.BlockSpec((RB, D), imap or (lambda i: (i, 0)))


def _wspec(r, c=D):
    return pl.BlockSpec((r, c), lambda i: (0, 0))


def _bspec():
    return pl.BlockSpec((D,), lambda i: (0,))


def _aspec():
    return pl.BlockSpec((NC, RB, D), lambda i: (0, i, 0))


_F32 = functools.partial(jnp.dot, preferred_element_type=jnp.float32)


def _stem_body(x_ref, dinv_ref, nout_ref, w1_ref, b1_ref, w2_ref, b2_ref,
               wg1_ref, h_ref, hh1_ref, fg1_ref):
    x = x_ref[...]
    h1 = jnp.maximum(_F32(x, w1_ref[...]) + b1_ref[...], 0.0)
    h = jnp.maximum(_F32(h1, w2_ref[...]) + b2_ref[...], 0.0)
    h_ref[...] = h
    hh1_ref[...] = h * dinv_ref[...]
    fg1_ref[...] = _F32(x * nout_ref[...], wg1_ref[...])


def _mid1_body(h_ref, a_ref, dinv_ref, l1_ref, hh2_ref):
    dinv = dinv_ref[...]
    l1 = h_ref[...] - (a_ref[0] + a_ref[1]) * dinv
    l1_ref[...] = l1
    hh2_ref[...] = l1 * dinv


def _band_body(h_ref, l1_ref, a2_ref, dinv_ref, w3_ref, b3_ref, w4_ref,
               b4_ref, out_ref):
    l1 = l1_ref[...]
    l2 = l1 - (a2_ref[0] + a2_ref[1]) * dinv_ref[...]
    w3 = w3_ref[...]
    wa = 3.0 * w3[0:128]
    wb = -3.0 * w3[0:128] + 3.0 * w3[128:256] + w3[384:512]
    wc = (0.75 * w3[0:128] - 1.5 * w3[128:256] + 0.75 * w3[256:384]
          + w3[512:640])
    acc = _F32(h_ref[...], wa) + _F32(l1, wb) + _F32(l2, wc) + b3_ref[...]
    out_ref[...] = _F32(jnp.maximum(acc, 0.0), w4_ref[...]) + b4_ref[...]


def _gmid_body(g1_ref, nin_ref, nout_ref, bg1_ref, wg2_ref, fg2_ref):
    emb1 = jnp.maximum((g1_ref[0] + g1_ref[1]) * nin_ref[...] + bg1_ref[...],
                       0.0)
    fg2_ref[...] = _F32(emb1 * nout_ref[...], wg2_ref[...])


def _gout_body(g2_ref, nin_ref, bg2_ref, emb_ref):
    emb_ref[...] = (g2_ref[0] + g2_ref[1]) * nin_ref[...] + bg2_ref[...]


def _nd(n_out=1):
    shp = jax.ShapeDtypeStruct((N_PAD, D), jnp.float32)
    return shp if n_out == 1 else (shp,) * n_out


def kernel(in_feat, edge_index, W1, b1, W2, b2, W3, b3, W4, b4,
           Wg1, bg1, Wg2, bg2):
    pad_idx = jnp.full((E_PAD - E_EDGES,), N_NODES, jnp.int32)
    srcp = jnp.concatenate([edge_index[0], pad_idx])
    dstp = jnp.concatenate([edge_index[1], pad_idx])
    x_pad = jnp.pad(in_feat, ((0, N_PAD - N_NODES), (0, 0)))
    zeros1 = jnp.zeros((N_PAD,), jnp.float32)
    zerosd = jnp.zeros((N_PAD, D), jnp.float32)
    w4p = jnp.pad(W4, ((0, 0), (0, D - W4.shape[1])))
    b4p = jnp.pad(b4, (0, D - b4.shape[0]))

    deg_raw, mdst = _sc_degree(srcp, dstp, zeros1)
    deg = deg_raw.reshape(NC, 3, N_PAD).sum(axis=0)

    scale = lax.rsqrt(jnp.maximum(deg, 1.0))
    dinvb = jnp.broadcast_to(scale[0][:, None], (N_PAD, D))
    noutb = jnp.broadcast_to(scale[1][:, None], (N_PAD, D))
    ninb = jnp.broadcast_to(scale[2][:, None], (N_PAD, D))

    h, hh1, fg1 = pl.pallas_call(
        _stem_body,
        grid=(GRID,),
        in_specs=[_rspec(), _rspec(), _rspec(), _wspec(D), _bspec(),
                  _wspec(D), _bspec(), _wspec(D)],
        out_specs=[_rspec(), _rspec(), _rspec()],
        out_shape=[_nd(), _nd(), _nd()],
    )(x_pad, dinvb, noutb, W1, b1, W2, b2, Wg1)

    a1 = _sc_scatter(hh1, srcp, dstp, zerosd)
    l1, hh2 = pl.pallas_call(
        _mid1_body,
        grid=(GRID,),
        in_specs=[_rspec(), _aspec(), _rspec()],
        out_specs=[_rspec(), _rspec()],
        out_shape=[_nd(), _nd()],
    )(h, a1, dinvb)

    a2 = _sc_scatter(hh2, srcp, dstp, zerosd)
    band = pl.pallas_call(
        _band_body,
        grid=(GRID,),
        in_specs=[_rspec(), _rspec(), _aspec(), _rspec(),
                  pl.BlockSpec((5 * D, D), lambda i: (0, 0)), _bspec(),
                  _wspec(D), _bspec()],
        out_specs=_rspec(),
        out_shape=_nd(),
    )(h, l1, a2, dinvb, W3, b3, w4p, b4p)

    g1 = _sc_scatter(fg1, srcp, mdst, zerosd)
    fg2 = pl.pallas_call(
        _gmid_body,
        grid=(GRID,),
        in_specs=[_aspec(), _rspec(), _rspec(), _bspec(), _wspec(D)],
        out_specs=_rspec(),
        out_shape=_nd(),
    )(g1, ninb, noutb, bg1, Wg2)

    g2 = _sc_scatter(fg2, srcp, mdst, zerosd)
    emb = pl.pallas_call(
        _gout_body,
        grid=(GRID,),
        in_specs=[_aspec(), _rspec(), _bspec()],
        out_specs=_rspec(),
        out_shape=_nd(),
    )(g2, ninb, bg2)

    return band[:N_NODES, :2], emb[:N_NODES]
